# baseline (speedup 1.0000x reference)
import functools

import jax
import jax.numpy as jnp
from jax import lax
from jax.experimental import pallas as pl
from jax.experimental.pallas import tpu as pltpu
from jax.experimental.pallas import tpu_sc as plsc

M, D, B, C = 100000, 256, 16384, 100
CP = 128
NC, NS = 2, 16
NW = NC * NS
RB = 160
NBLK = M // RB
TPW = (NBLK + NW - 1) // NW
L = 16
NG = RB // L
NGF = 2
NSEC = TPW + 4
NIT = (NSEC + 5) // 6

_mesh = plsc.VectorSubcoreMesh(core_axis_name="c", subcore_axis_name="s",
                               num_cores=NC, num_subcores=NS)


@functools.partial(
    pl.kernel,
    out_type=(
        jax.ShapeDtypeStruct((M,), jnp.int32),
        jax.ShapeDtypeStruct((M,), jnp.float32),
        jax.ShapeDtypeStruct((M, C), jnp.float32),
    ),
    mesh=_mesh,
    compiler_params=pltpu.CompilerParams(needs_layout_passes=False),
    scratch_types=[
        pltpu.VMEM((3, RB, C), jnp.float32),
        pltpu.VMEM((RB,), jnp.int32),
        pltpu.VMEM((RB,), jnp.int32),
        pltpu.VMEM((RB,), jnp.int32),
        pltpu.VMEM((RB,), jnp.float32),
        pltpu.VMEM((RB,), jnp.float32),
        pltpu.VMEM((RB,), jnp.float32),
        pltpu.VMEM((RB,), jnp.int32),
        pltpu.VMEM((RB,), jnp.int32),
        pltpu.VMEM((RB,), jnp.int32),
        pltpu.VMEM((RB + L,), jnp.int32),
        pltpu.VMEM((RB + L,), jnp.int32),
        pltpu.VMEM((RB + L,), jnp.int32),
        pltpu.VMEM((RB + L,), jnp.int32),
        pltpu.VMEM((2, NGF, L, D), jnp.float32),
        pltpu.VMEM((2, NGF, L, CP), jnp.float32),
        pltpu.VMEM((L,), jnp.int32),
        pltpu.VMEM((L,), jnp.int32),
        pltpu.VMEM((L,), jnp.int32),
        pltpu.VMEM((L,), jnp.int32),
        pltpu.VMEM((L,), jnp.float32),
        pltpu.VMEM((L,), jnp.float32),
        pltpu.VMEM((L,), jnp.float32),
        pltpu.VMEM((L,), jnp.float32),
        pltpu.VMEM((L,), jnp.int32),
        pltpu.VMEM((L,), jnp.int32),
        pltpu.VMEM((L,), jnp.int32),
        pltpu.VMEM((L,), jnp.int32),
        pltpu.VMEM((L, D), jnp.float32),
        pltpu.VMEM((L, CP), jnp.float32),
        pltpu.VMEM((L,), jnp.int32),
        pltpu.VMEM((L,), jnp.float32),
        pltpu.VMEM((L,), jnp.int32),
        pltpu.SMEM((8,), jnp.int32),
        pltpu.SemaphoreType.DMA,
        pltpu.SemaphoreType.DMA,
        pltpu.SemaphoreType.DMA,
        pltpu.SemaphoreType.DMA,
        pltpu.SemaphoreType.DMA,
        pltpu.SemaphoreType.DMA,
        pltpu.SemaphoreType.DMA,
        pltpu.SemaphoreType.DMA,
        pltpu.SemaphoreType.DMA,
        pltpu.SemaphoreType.DMA,
        pltpu.SemaphoreType.DMA,
    ],
)
def _patch(bx_ref, by_in, ents_in, logits_in, x, y, ents, logits_pad, owner,
           out_by, out_ents, out_logits,
           lgb, byb0, byb1, byb2, entsb0, entsb1, entsb2,
           ownb0, ownb1, ownb2, tgt_l0, tgt_l1, src_l0, src_l1, xs, ls,
           ys00, ys01, ys10, ys11,
           es00, es01, es10, es11,
           tb00, tb01, tb10, tb11,
           sxs, sls, sys_, ses, stabs, cnts,
           bsem0, bsem1, bsem2, wsem0, wsem1, wsem2,
           gsem0, gsem1, ssem0, ssem1, slsem):
    w = lax.axis_index("s") * NC + lax.axis_index("c")
    iota = lax.broadcasted_iota(jnp.int32, (L,), 0)
    byb = (byb0, byb1, byb2)
    entsb = (entsb0, entsb1, entsb2)
    ownb = (ownb0, ownb1, ownb2)
    tgt_l = (tgt_l0, tgt_l1)
    src_l = (src_l0, src_l1)
    ys = ((ys00, ys01), (ys10, ys11))
    es = ((es00, es01), (es10, es11))
    tabs = ((tb00, tb01), (tb10, tb11))
    bsem = (bsem0, bsem1, bsem2)
    wsem = (wsem0, wsem1, wsem2)
    gsem = (gsem0, gsem1)
    ssem = (ssem0, ssem1)

    def in_copies(t, s):
        r0 = (w + NW * t) * RB
        return (
            pltpu.make_async_copy(logits_in.at[pl.ds(r0, RB)], lgb.at[s],
                                  bsem[s]),
            pltpu.make_async_copy(by_in.at[pl.ds(r0, RB)], byb[s],
                                  bsem[s]),
            pltpu.make_async_copy(ents_in.at[pl.ds(r0, RB)], entsb[s],
                                  bsem[s]),
            pltpu.make_async_copy(owner.at[pl.ds(r0, RB)], ownb[s],
                                  bsem[s]),
        )

    def out_copies(t, s):
        r0 = (w + NW * t) * RB
        return (
            pltpu.make_async_copy(lgb.at[s], out_logits.at[pl.ds(r0, RB)],
                                  wsem[s]),
            pltpu.make_async_copy(byb[s], out_by.at[pl.ds(r0, RB)],
                                  wsem[s]),
            pltpu.make_async_copy(entsb[s], out_ents.at[pl.ds(r0, RB)],
                                  wsem[s]),
        )

    def gather_group(p, g):
        sl = src_l[p][pl.ds(g * L, L)]
        return (
            pltpu.make_async_copy(x.at[sl], xs.at[p].at[g], gsem[p]),
            pltpu.make_async_copy(logits_pad.at[sl], ls.at[p].at[g],
                                  gsem[p]),
            pltpu.make_async_copy(y.at[sl], ys[p][g], gsem[p]),
            pltpu.make_async_copy(ents.at[sl], es[p][g], gsem[p]),
        )

    def front(t, s, p):
        bk = w + NW * t

        @pl.when(bk < NBLK)
        def _():
            for cpy in in_copies(t, s):
                cpy.wait()
            cnt = jnp.int32(0)
            for g in range(NG):
                ov = ownb[s][pl.ds(g * L, L)]
                mask = ov >= 0
                pos = iota + (g * L)
                cpos = plsc.cumsum(mask.astype(jnp.int32)) - 1 + cnt
                plsc.store_scatter(tgt_l[p], [cpos], pos, mask=mask)
                plsc.store_scatter(src_l[p], [cpos], ov, mask=mask)
                cnt = cnt + jnp.sum(mask.astype(jnp.int32))

            @pl.when(cnt > 0)
            def _():
                last = jnp.full((L,), 1, jnp.int32) * (cnt - 1)
                tv = plsc.load_gather(tgt_l[p], [last])
                sv = plsc.load_gather(src_l[p], [last])
                tgt_l[p][pl.ds(cnt, L)] = tv
                src_l[p][pl.ds(cnt, L)] = sv

            def sdrain(g, c):
                pltpu.make_async_copy(xs.at[p].at[0],
                                      bx_ref.at[tabs[p][0]],
                                      ssem[p]).wait()
                return c

            lax.fori_loop(0, cnts[2 + p], sdrain, 0)
            cnts[2 + p] = 0

            cnts[p] = cnt
            ngrp = (cnt + (L - 1)) // L
            for g in range(NGF):
                @pl.when(g < ngrp)
                def _(g=g):
                    for cpy in gather_group(p, g):
                        cpy.start()

    def back(t, s, p):
        bk = w + NW * t

        @pl.when(bk < NBLK)
        def _():
            r0 = bk * RB
            cnt = cnts[p]
            ngrp = (cnt + (L - 1)) // L
            nfast = jnp.minimum(ngrp, NGF)
            for g in range(NGF):
                @pl.when(g < ngrp)
                def _(g=g):
                    for cpy in gather_group(p, g):
                        cpy.wait()
                    tl = tgt_l[p][pl.ds(g * L, L)]
                    tabs[p][g][...] = tl + r0
                    pltpu.make_async_copy(xs.at[p].at[g],
                                          bx_ref.at[tabs[p][g]],
                                          ssem[p]).start()
                    for c in range(C):
                        col = jnp.full((L,), c, jnp.int32)
                        v = plsc.load_gather(ls.at[p].at[g], [iota, col])
                        plsc.store_scatter(lgb.at[s], [tl, col], v)
                    plsc.store_scatter(byb[s], [tl], ys[p][g][...])
                    plsc.store_scatter(entsb[s], [tl], es[p][g][...])

            cnts[2 + p] = nfast

            def slow(g, c):
                sl = src_l[p][pl.ds(g * L, L)]
                pltpu.make_async_copy(x.at[sl], sxs, slsem).start()
                pltpu.make_async_copy(logits_pad.at[sl], sls, slsem).start()
                pltpu.make_async_copy(y.at[sl], sys_, slsem).start()
                pltpu.make_async_copy(ents.at[sl], ses, slsem).start()
                pltpu.make_async_copy(x.at[sl], sxs, slsem).wait()
                pltpu.make_async_copy(logits_pad.at[sl], sls, slsem).wait()
                pltpu.make_async_copy(y.at[sl], sys_, slsem).wait()
                pltpu.make_async_copy(ents.at[sl], ses, slsem).wait()
                tl = tgt_l[p][pl.ds(g * L, L)]
                stabs[...] = tl + r0
                pltpu.make_async_copy(sxs, bx_ref.at[stabs], slsem).start()
                for c2 in range(C):
                    col = jnp.full((L,), c2, jnp.int32)
                    v = plsc.load_gather(sls, [iota, col])
                    plsc.store_scatter(lgb.at[s], [tl, col], v)
                plsc.store_scatter(byb[s], [tl], sys_[...])
                plsc.store_scatter(entsb[s], [tl], ses[...])
                pltpu.make_async_copy(sxs, bx_ref.at[stabs], slsem).wait()
                return c

            lax.fori_loop(NGF, ngrp, slow, 0)

            for cpy in out_copies(t, s):
                cpy.start()

    def section(t, a):
        s = a % 3
        p = a % 2
        bk_m2 = w + NW * (t - 2)

        @pl.when((t >= 2) & (bk_m2 < NBLK))
        def _():
            for cpy in out_copies(t - 2, (a + 1) % 3):
                cpy.wait()

        bk_p1 = w + NW * (t + 1)

        @pl.when(bk_p1 < NBLK)
        def _():
            for cpy in in_copies(t + 1, (a + 1) % 3):
                cpy.start()

        front(t, s, p)

        @pl.when(t >= 1)
        def _():
            back(t - 1, (a + 2) % 3, (a + 1) % 2)

    cnts[2] = jnp.int32(0)
    cnts[3] = jnp.int32(0)
    for cpy in in_copies(0, 0):
        cpy.start()

    def body(i, c):
        t0 = i * 6
        for a in range(6):
            section(t0 + a, a)
        return c

    lax.fori_loop(0, NIT, body, 0)

    for p in range(2):
        def fdrain(g, c, p=p):
            pltpu.make_async_copy(xs.at[p].at[0], bx_ref.at[tabs[p][0]],
                                  ssem[p]).wait()
            return c

        lax.fori_loop(0, cnts[2 + p], fdrain, 0)


def kernel(bx, by_buf, ents_buf, logits_buf, x, y, ents, logits, idx):
    js = jnp.arange(B, dtype=jnp.int32)
    owner = jnp.full((M,), -1, jnp.int32).at[idx].set(js)
    logits_pad = jnp.pad(logits, ((0, 0), (0, CP - C)))
    rbx = jax.new_ref(bx)
    out_by, out_ents, out_logits = _patch(
        rbx, by_buf, ents_buf, logits_buf, x, y, ents, logits_pad, owner)
    return jax.freeze(rbx), out_by, out_ents, out_logits


# speedup vs baseline: 1.0534x; 1.0534x over previous
import functools

import jax
import jax.numpy as jnp
from jax import lax
from jax.experimental import pallas as pl
from jax.experimental.pallas import tpu as pltpu
from jax.experimental.pallas import tpu_sc as plsc

M, D, B, C = 100000, 256, 16384, 100
CP = 128
NC, NS = 2, 16
NW = NC * NS
RB = 200
NBLK = M // RB
TPW = (NBLK + NW - 1) // NW
L = 16
NG = (RB + L - 1) // L
RBP = NG * L
NGF = 3
NSEC = TPW + 4
NIT = (NSEC + 5) // 6

_mesh = plsc.VectorSubcoreMesh(core_axis_name="c", subcore_axis_name="s",
                               num_cores=NC, num_subcores=NS)


@functools.partial(
    pl.kernel,
    out_type=(
        jax.ShapeDtypeStruct((M,), jnp.int32),
        jax.ShapeDtypeStruct((M,), jnp.float32),
        jax.ShapeDtypeStruct((M, C), jnp.float32),
    ),
    mesh=_mesh,
    compiler_params=pltpu.CompilerParams(needs_layout_passes=False),
    scratch_types=[
        pltpu.VMEM((3, RB, C), jnp.float32),
        pltpu.VMEM((RBP,), jnp.int32),
        pltpu.VMEM((RBP,), jnp.int32),
        pltpu.VMEM((RBP,), jnp.int32),
        pltpu.VMEM((RBP,), jnp.float32),
        pltpu.VMEM((RBP,), jnp.float32),
        pltpu.VMEM((RBP,), jnp.float32),
        pltpu.VMEM((RBP,), jnp.int32),
        pltpu.VMEM((RBP,), jnp.int32),
        pltpu.VMEM((RBP,), jnp.int32),
        pltpu.VMEM((RBP + L,), jnp.int32),
        pltpu.VMEM((RBP + L,), jnp.int32),
        pltpu.VMEM((RBP + L,), jnp.int32),
        pltpu.VMEM((RBP + L,), jnp.int32),
        pltpu.VMEM((2, NGF, L, D), jnp.float32),
        pltpu.VMEM((2, NGF, L, CP), jnp.float32),
        pltpu.VMEM((L,), jnp.int32),
        pltpu.VMEM((L,), jnp.int32),
        pltpu.VMEM((L,), jnp.int32),
        pltpu.VMEM((L,), jnp.int32),
        pltpu.VMEM((L,), jnp.int32),
        pltpu.VMEM((L,), jnp.int32),
        pltpu.VMEM((L,), jnp.float32),
        pltpu.VMEM((L,), jnp.float32),
        pltpu.VMEM((L,), jnp.float32),
        pltpu.VMEM((L,), jnp.float32),
        pltpu.VMEM((L,), jnp.float32),
        pltpu.VMEM((L,), jnp.float32),
        pltpu.VMEM((L,), jnp.int32),
        pltpu.VMEM((L,), jnp.int32),
        pltpu.VMEM((L,), jnp.int32),
        pltpu.VMEM((L,), jnp.int32),
        pltpu.VMEM((L,), jnp.int32),
        pltpu.VMEM((L,), jnp.int32),
        pltpu.VMEM((L, D), jnp.float32),
        pltpu.VMEM((L, CP), jnp.float32),
        pltpu.VMEM((L,), jnp.int32),
        pltpu.VMEM((L,), jnp.float32),
        pltpu.VMEM((L,), jnp.int32),
        pltpu.SMEM((8,), jnp.int32),
        pltpu.SemaphoreType.DMA,
        pltpu.SemaphoreType.DMA,
        pltpu.SemaphoreType.DMA,
        pltpu.SemaphoreType.DMA,
        pltpu.SemaphoreType.DMA,
        pltpu.SemaphoreType.DMA,
        pltpu.SemaphoreType.DMA,
        pltpu.SemaphoreType.DMA,
        pltpu.SemaphoreType.DMA,
        pltpu.SemaphoreType.DMA,
        pltpu.SemaphoreType.DMA,
    ],
)
def _patch(bx_ref, by_in, ents_in, logits_in, x, y, ents, logits_pad, owner,
           out_by, out_ents, out_logits,
           lgb, byb0, byb1, byb2, entsb0, entsb1, entsb2,
           ownb0, ownb1, ownb2, tgt_l0, tgt_l1, src_l0, src_l1, xs, ls,
           ys00, ys01, ys02, ys10, ys11, ys12,
           es00, es01, es02, es10, es11, es12,
           tb00, tb01, tb02, tb10, tb11, tb12,
           sxs, sls, sys_, ses, stabs, cnts,
           bsem0, bsem1, bsem2, wsem0, wsem1, wsem2,
           gsem0, gsem1, ssem0, ssem1, slsem):
    w = lax.axis_index("s") * NC + lax.axis_index("c")
    iota = lax.broadcasted_iota(jnp.int32, (L,), 0)
    byb = (byb0, byb1, byb2)
    entsb = (entsb0, entsb1, entsb2)
    ownb = (ownb0, ownb1, ownb2)
    tgt_l = (tgt_l0, tgt_l1)
    src_l = (src_l0, src_l1)
    ys = ((ys00, ys01, ys02), (ys10, ys11, ys12))
    es = ((es00, es01, es02), (es10, es11, es12))
    tabs = ((tb00, tb01, tb02), (tb10, tb11, tb12))
    bsem = (bsem0, bsem1, bsem2)
    wsem = (wsem0, wsem1, wsem2)
    gsem = (gsem0, gsem1)
    ssem = (ssem0, ssem1)

    def in_copies(t, s):
        r0 = (w + NW * t) * RB
        return (
            pltpu.make_async_copy(logits_in.at[pl.ds(r0, RB)], lgb.at[s],
                                  bsem[s]),
            pltpu.make_async_copy(by_in.at[pl.ds(r0, RB)], byb[s].at[pl.ds(0, RB)],
                                  bsem[s]),
            pltpu.make_async_copy(ents_in.at[pl.ds(r0, RB)], entsb[s].at[pl.ds(0, RB)],
                                  bsem[s]),
            pltpu.make_async_copy(owner.at[pl.ds(r0, RB)], ownb[s].at[pl.ds(0, RB)],
                                  bsem[s]),
        )

    def out_copies(t, s):
        r0 = (w + NW * t) * RB
        return (
            pltpu.make_async_copy(lgb.at[s], out_logits.at[pl.ds(r0, RB)],
                                  wsem[s]),
            pltpu.make_async_copy(byb[s].at[pl.ds(0, RB)], out_by.at[pl.ds(r0, RB)],
                                  wsem[s]),
            pltpu.make_async_copy(entsb[s].at[pl.ds(0, RB)], out_ents.at[pl.ds(r0, RB)],
                                  wsem[s]),
        )

    def gather_group(p, g):
        sl = src_l[p][pl.ds(g * L, L)]
        return (
            pltpu.make_async_copy(x.at[sl], xs.at[p].at[g], gsem[p]),
            pltpu.make_async_copy(logits_pad.at[sl], ls.at[p].at[g],
                                  gsem[p]),
            pltpu.make_async_copy(y.at[sl], ys[p][g], gsem[p]),
            pltpu.make_async_copy(ents.at[sl], es[p][g], gsem[p]),
        )

    def front(t, s, p):
        bk = w + NW * t

        @pl.when(bk < NBLK)
        def _():
            for cpy in in_copies(t, s):
                cpy.wait()
            cnt = jnp.int32(0)
            for g in range(NG):
                ov = ownb[s][pl.ds(g * L, L)]
                pos = iota + (g * L)
                mask = ov >= 0
                if (g + 1) * L > RB:
                    mask = mask & (pos < RB)
                cpos = plsc.cumsum(mask.astype(jnp.int32)) - 1 + cnt
                plsc.store_scatter(tgt_l[p], [cpos], pos, mask=mask)
                plsc.store_scatter(src_l[p], [cpos], ov, mask=mask)
                cnt = cnt + jnp.sum(mask.astype(jnp.int32))

            @pl.when(cnt > 0)
            def _():
                last = jnp.full((L,), 1, jnp.int32) * (cnt - 1)
                tv = plsc.load_gather(tgt_l[p], [last])
                sv = plsc.load_gather(src_l[p], [last])
                tgt_l[p][pl.ds(cnt, L)] = tv
                src_l[p][pl.ds(cnt, L)] = sv

            def sdrain(g, c):
                pltpu.make_async_copy(xs.at[p].at[0],
                                      bx_ref.at[tabs[p][0]],
                                      ssem[p]).wait()
                return c

            lax.fori_loop(0, cnts[2 + p], sdrain, 0)
            cnts[2 + p] = 0

            cnts[p] = cnt
            ngrp = (cnt + (L - 1)) // L
            for g in range(NGF):
                @pl.when(g < ngrp)
                def _(g=g):
                    for cpy in gather_group(p, g):
                        cpy.start()

    def back(t, s, p):
        bk = w + NW * t

        @pl.when(bk < NBLK)
        def _():
            r0 = bk * RB
            cnt = cnts[p]
            ngrp = (cnt + (L - 1)) // L
            nfast = jnp.minimum(ngrp, NGF)
            for g in range(NGF):
                @pl.when(g < ngrp)
                def _(g=g):
                    for cpy in gather_group(p, g):
                        cpy.wait()
                    tl = tgt_l[p][pl.ds(g * L, L)]
                    tabs[p][g][...] = tl + r0
                    pltpu.make_async_copy(xs.at[p].at[g],
                                          bx_ref.at[tabs[p][g]],
                                          ssem[p]).start()
                    def colbody(cb, carry, g=g):
                        for dc in range(4):
                            col = jnp.full((L,), 1, jnp.int32) * (cb * 4 + dc)
                            v = plsc.load_gather(ls.at[p].at[g], [iota, col])
                            plsc.store_scatter(lgb.at[s], [tl, col], v)
                        return carry

                    lax.fori_loop(0, C // 4, colbody, 0)
                    plsc.store_scatter(byb[s], [tl], ys[p][g][...])
                    plsc.store_scatter(entsb[s], [tl], es[p][g][...])

            cnts[2 + p] = nfast

            def slow(g, c):
                sl = src_l[p][pl.ds(g * L, L)]
                pltpu.make_async_copy(x.at[sl], sxs, slsem).start()
                pltpu.make_async_copy(logits_pad.at[sl], sls, slsem).start()
                pltpu.make_async_copy(y.at[sl], sys_, slsem).start()
                pltpu.make_async_copy(ents.at[sl], ses, slsem).start()
                pltpu.make_async_copy(x.at[sl], sxs, slsem).wait()
                pltpu.make_async_copy(logits_pad.at[sl], sls, slsem).wait()
                pltpu.make_async_copy(y.at[sl], sys_, slsem).wait()
                pltpu.make_async_copy(ents.at[sl], ses, slsem).wait()
                tl = tgt_l[p][pl.ds(g * L, L)]
                stabs[...] = tl + r0
                pltpu.make_async_copy(sxs, bx_ref.at[stabs], slsem).start()
                def colbody2(cb, carry):
                    for dc in range(4):
                        col = jnp.full((L,), 1, jnp.int32) * (cb * 4 + dc)
                        v = plsc.load_gather(sls, [iota, col])
                        plsc.store_scatter(lgb.at[s], [tl, col], v)
                    return carry

                lax.fori_loop(0, C // 4, colbody2, 0)
                plsc.store_scatter(byb[s], [tl], sys_[...])
                plsc.store_scatter(entsb[s], [tl], ses[...])
                pltpu.make_async_copy(sxs, bx_ref.at[stabs], slsem).wait()
                return c

            lax.fori_loop(NGF, ngrp, slow, 0)

            for cpy in out_copies(t, s):
                cpy.start()

    def section(t, a):
        s = a % 3
        p = a % 2
        bk_m2 = w + NW * (t - 2)

        @pl.when((t >= 2) & (bk_m2 < NBLK))
        def _():
            for cpy in out_copies(t - 2, (a + 1) % 3):
                cpy.wait()

        bk_p1 = w + NW * (t + 1)

        @pl.when(bk_p1 < NBLK)
        def _():
            for cpy in in_copies(t + 1, (a + 1) % 3):
                cpy.start()

        front(t, s, p)

        @pl.when(t >= 1)
        def _():
            back(t - 1, (a + 2) % 3, (a + 1) % 2)

    cnts[2] = jnp.int32(0)
    cnts[3] = jnp.int32(0)
    for cpy in in_copies(0, 0):
        cpy.start()

    def body(i, c):
        t0 = i * 6
        for a in range(6):
            section(t0 + a, a)
        return c

    lax.fori_loop(0, NIT, body, 0)

    for p in range(2):
        def fdrain(g, c, p=p):
            pltpu.make_async_copy(xs.at[p].at[0], bx_ref.at[tabs[p][0]],
                                  ssem[p]).wait()
            return c

        lax.fori_loop(0, cnts[2 + p], fdrain, 0)


def kernel(bx, by_buf, ents_buf, logits_buf, x, y, ents, logits, idx):
    js = jnp.arange(B, dtype=jnp.int32)
    owner = jnp.full((M,), -1, jnp.int32).at[idx].set(js)
    logits_pad = jnp.pad(logits, ((0, 0), (0, CP - C)))
    rbx = jax.new_ref(bx)
    out_by, out_ents, out_logits = _patch(
        rbx, by_buf, ents_buf, logits_buf, x, y, ents, logits_pad, owner)
    return jax.freeze(rbx), out_by, out_ents, out_logits
